# baseline (speedup 1.0000x reference)
import functools

import jax
import jax.numpy as jnp
from jax import lax
from jax.experimental import pallas as pl
from jax.experimental.pallas import tpu as pltpu
from jax.experimental.pallas import tpu_sc as plsc

N = 10000
E = 320000
D = 128
G = 64

NC = 2
NS = 16
NW = NC * NS

EC = 50
ER = E // EC
RPW = ER // NW
KC = 8
TO = RPW // KC
EPW = E // NW
RPS0 = 632
RPS1 = N - 15 * RPS0

def _mesh():
  return plsc.VectorSubcoreMesh(
      core_axis_name="c", subcore_axis_name="s", num_cores=NC, num_subcores=NS
  )


def _sc_degree(dst3):
  zeros = jnp.zeros((EPW,), jnp.float32)

  @functools.partial(
      pl.kernel,
      out_type=jax.ShapeDtypeStruct((NW, N), jnp.float32),
      mesh=_mesh(),
      compiler_params=pltpu.CompilerParams(use_tc_tiling_on_sc=False,
                                           needs_layout_passes=False),
      scratch_types=[
          pltpu.VMEM((EPW,), jnp.int32),
          pltpu.VMEM((N,), jnp.float32),
      ],
  )
  def k(dst_h, zeros_h, out_h, idx_v, acc_v):
    c = lax.axis_index("c")
    s = lax.axis_index("s")
    w = c * NS + s
    pltpu.sync_copy(zeros_h, acc_v)
    pltpu.sync_copy(dst_h.at[w], idx_v)
    ones16 = jnp.ones((16,), jnp.float32)

    @pl.loop(0, EPW // 16)
    def _r(r):
      iv = idx_v[pl.ds(r * 16, 16)]
      plsc.addupdate_scatter(acc_v, [iv], ones16)

    pltpu.sync_copy(acc_v, out_h.at[w])

  return k(dst3, zeros)


def _sc_scatter(y, src2, dst2):
  zeros = jnp.zeros((RPS0, D), jnp.float32)

  @functools.partial(
      pl.kernel,
      out_type=jax.ShapeDtypeStruct((NC, N, D), jnp.float32),
      mesh=_mesh(),
      scratch_types=[
          pltpu.VMEM((2, KC, EC), jnp.int32),
          pltpu.VMEM((2, KC, EC), jnp.int32),
          pltpu.VMEM((EC, D), jnp.float32),
          pltpu.VMEM((EC, D), jnp.float32),
          pltpu.VMEM((EC, D), jnp.float32),
          pltpu.VMEM((EC, D), jnp.float32),
          pltpu.VMEM_SHARED((N, D), jnp.float32),
          pltpu.SemaphoreType.DMA,
          pltpu.SemaphoreType.DMA,
          pltpu.SemaphoreType.DMA,
          pltpu.SemaphoreType.DMA,
          pltpu.SemaphoreType.DMA,
          pltpu.SemaphoreType.DMA,
          pltpu.SemaphoreType.DMA,
          pltpu.SemaphoreType.DMA,
          pltpu.SemaphoreType.DMA,
          pltpu.SemaphoreType.DMA,
      ],
  )
  def k(y_h, src_h, dst_h, zeros_h, out_h, sidx_v, didx_v,
        rows0_v, rows1_v, rows2_v, rows3_v, acc_sh,
        gsem0, gsem1, gsem2, gsem3, ssem0, ssem1, ssem2, ssem3,
        isem0, isem1):
    c = lax.axis_index("c")
    s = lax.axis_index("s")
    w = c * NS + s

    @pl.when(s < NS - 1)
    def _():
      pltpu.sync_copy(zeros_h.at[pl.ds(0, RPS0)], acc_sh.at[pl.ds(s * RPS0, RPS0)])
    @pl.when(s == NS - 1)
    def _():
      pltpu.sync_copy(zeros_h.at[pl.ds(0, RPS1)], acc_sh.at[pl.ds(15 * RPS0, RPS1)])
    plsc.subcore_barrier()

    base = w * RPW
    rows = (rows0_v, rows1_v, rows2_v, rows3_v)
    gsem = (gsem0, gsem1, gsem2, gsem3)
    ssem = (ssem0, ssem1, ssem2, ssem3)
    isem = (isem0, isem1)
    LA = 3

    def prefetch_idx(t, slot):
      pltpu.async_copy(src_h.at[pl.ds(base + t * KC, KC)],
                       sidx_v.at[slot], isem[slot])
      pltpu.async_copy(dst_h.at[pl.ds(base + t * KC, KC)],
                       didx_v.at[slot], isem[slot])

    def wait_idx(slot):
      pltpu.make_async_copy(src_h.at[pl.ds(base, KC)], sidx_v.at[slot],
                            isem[slot]).wait()
      pltpu.make_async_copy(dst_h.at[pl.ds(base, KC)], didx_v.at[slot],
                            isem[slot]).wait()

    def process(t, slot):
      @pl.when(t + 1 < TO)
      def _():
        prefetch_idx(t + 1, slot ^ 1)
      wait_idx(slot)
      si = sidx_v.at[slot]
      di = didx_v.at[slot]
      g = [None] * KC
      sd = [None] * KC
      for jj in range(LA):
        g[jj] = pltpu.async_copy(y_h.at[si.at[jj]], rows[jj & 3], gsem[jj & 3])
      for j in range(KC):
        if j + LA < KC:
          if j >= 1:
            sd[j - 1].wait()
          g[j + LA] = pltpu.async_copy(
              y_h.at[si.at[j + LA]], rows[(j + LA) & 3], gsem[(j + LA) & 3])
        g[j].wait()
        sd[j] = pltpu.async_copy(
            rows[j & 3], acc_sh.at[di.at[j]], ssem[j & 3], add=True)
      for j in range(KC - LA - 1, KC):
        sd[j].wait()

    prefetch_idx(0, 0)

    @pl.loop(0, TO // 2)
    def _outer(u):
      process(2 * u, 0)
      process(2 * u + 1, 1)

    if TO % 2 == 1:
      process(TO - 1, 0)

    plsc.subcore_barrier()

    @pl.when(s < NS - 1)
    def _():
      pltpu.sync_copy(acc_sh.at[pl.ds(s * RPS0, RPS0)],
                      out_h.at[c, pl.ds(s * RPS0, RPS0)])
    @pl.when(s == NS - 1)
    def _():
      pltpu.sync_copy(acc_sh.at[pl.ds(15 * RPS0, RPS1)],
                      out_h.at[c, pl.ds(15 * RPS0, RPS1)])

  return k(y, src2, dst2, zeros)


BN = 1000
NB = N // BN


def _tc_dis(degp):
  def body(d_ref, dis_ref):
    ones = jnp.ones((NW, 16), jnp.float32)
    deg = lax.dot_general(d_ref[...], ones, (((0,), (0,)), ((), ())),
                          preferred_element_type=jnp.float32) + 1.0
    dis_ref[...] = lax.rsqrt(deg)

  return pl.pallas_call(
      body,
      grid=(),
      in_specs=[pl.BlockSpec((NW, N), lambda: (0, 0))],
      out_specs=pl.BlockSpec((N, 16), lambda: (0, 0)),
      out_shape=jax.ShapeDtypeStruct((N, 16), jnp.float32),
  )(degp)


def _tc_y1(x, W1, dis16):
  def body(x_ref, w_ref, dis_ref, y_ref):
    xw = jnp.dot(x_ref[...], w_ref[...], preferred_element_type=jnp.float32)
    y_ref[...] = xw * dis_ref[:, 0:1]

  return pl.pallas_call(
      body,
      grid=(NB,),
      in_specs=[
          pl.BlockSpec((BN, D), lambda i: (i, 0)),
          pl.BlockSpec((D, D), lambda i: (0, 0)),
          pl.BlockSpec((BN, 16), lambda i: (i, 0)),
      ],
      out_specs=pl.BlockSpec((BN, D), lambda i: (i, 0)),
      out_shape=jax.ShapeDtypeStruct((N, D), jnp.float32),
  )(x, W1, dis16)


def _tc_mid(accp, y, dis16, b, W2):
  def body(a_ref, y_ref, dis_ref, b_ref, w_ref, o_ref):
    dis = dis_ref[:, 0:1]
    h = jnp.maximum(dis * (a_ref[0] + a_ref[1] + y_ref[...]) + b_ref[...], 0.0)
    hw = jnp.dot(h, w_ref[...], preferred_element_type=jnp.float32)
    o_ref[...] = hw * dis

  return pl.pallas_call(
      body,
      grid=(NB,),
      in_specs=[
          pl.BlockSpec((NC, BN, D), lambda i: (0, i, 0)),
          pl.BlockSpec((BN, D), lambda i: (i, 0)),
          pl.BlockSpec((BN, 16), lambda i: (i, 0)),
          pl.BlockSpec((1, D), lambda i: (0, 0)),
          pl.BlockSpec((D, D), lambda i: (0, 0)),
      ],
      out_specs=pl.BlockSpec((BN, D), lambda i: (i, 0)),
      out_shape=jax.ShapeDtypeStruct((N, D), jnp.float32),
  )(accp, y, dis16, b, W2)


def _tc_final(accp, y, dis16, b, brow, bcol, Wl1, bl1, Wl2, bl2):
  def body(a_ref, y_ref, dis_ref, b_ref, br_ref, bc_ref,
           wl1_ref, bl1_ref, wl2_ref, bl2_ref, o_ref,
           gsum, gmax, cnt):
    i = pl.program_id(0)

    @pl.when(i == 0)
    def _():
      gsum[...] = jnp.zeros_like(gsum)
      cnt[...] = jnp.zeros_like(cnt)
      gmax[...] = jnp.full_like(gmax, -jnp.inf)

    dis = dis_ref[:, 0:1]
    h = jnp.maximum(dis * (a_ref[0] + a_ref[1] + y_ref[...]) + b_ref[...], 0.0)

    brv = br_ref[0]
    oh = (lax.broadcasted_iota(jnp.int32, (G, BN), 0)
          == jnp.broadcast_to(brv, (G, BN))).astype(jnp.float32)
    gsum[...] += jnp.dot(oh, h, preferred_element_type=jnp.float32)
    cnt[...] += jnp.broadcast_to(jnp.sum(oh, axis=1, keepdims=True), (G, D))

    bcv = bc_ref[...]
    bmin = jnp.min(bcv)
    bmax = jnp.max(bcv)
    for g in range(G):
      @pl.when(jnp.logical_and(bmin <= g, g <= bmax))
      def _(g=g):
        m = jnp.max(jnp.where(bcv == g, h, -jnp.inf), axis=0, keepdims=True)
        gmax[g:g + 1, :] = jnp.maximum(gmax[g:g + 1, :], m)

    @pl.when(i == NB - 1)
    def _():
      mean = gsum[...] / jnp.maximum(cnt[...], 1.0)
      pooled = jnp.concatenate([gmax[...], mean], axis=1)
      h1 = jnp.maximum(
          jnp.dot(pooled, wl1_ref[...], preferred_element_type=jnp.float32)
          + bl1_ref[...], 0.0)
      o_ref[...] = (jnp.dot(h1, wl2_ref[...],
                            preferred_element_type=jnp.float32)
                    + bl2_ref[...])

  return pl.pallas_call(
      body,
      grid=(NB,),
      in_specs=[
          pl.BlockSpec((NC, BN, D), lambda i: (0, i, 0)),
          pl.BlockSpec((BN, D), lambda i: (i, 0)),
          pl.BlockSpec((BN, 16), lambda i: (i, 0)),
          pl.BlockSpec((1, D), lambda i: (0, 0)),
          pl.BlockSpec((1, 1, BN), lambda i: (i, 0, 0)),
          pl.BlockSpec((BN, 1), lambda i: (i, 0)),
          pl.BlockSpec((2 * D, 512), lambda i: (0, 0)),
          pl.BlockSpec((1, 512), lambda i: (0, 0)),
          pl.BlockSpec((512, D), lambda i: (0, 0)),
          pl.BlockSpec((1, D), lambda i: (0, 0)),
      ],
      out_specs=pl.BlockSpec((G, D), lambda i: (0, 0)),
      out_shape=jax.ShapeDtypeStruct((G, D), jnp.float32),
      scratch_shapes=[
          pltpu.VMEM((G, D), jnp.float32),
          pltpu.VMEM((G, D), jnp.float32),
          pltpu.VMEM((G, D), jnp.float32),
      ],
  )(accp, y, dis16, b, brow, bcol, Wl1, bl1, Wl2, bl2)


def kernel(x, edge_index, batch, W1, b1, W2, b2, Wl1, bl1, Wl2, bl2):
  src2 = edge_index[0].reshape(ER, EC)
  dst2 = edge_index[1].reshape(ER, EC)
  dst3 = edge_index[1].reshape(NW, EPW)

  degp = _sc_degree(dst3)
  dis16 = _tc_dis(degp)
  y1 = _tc_y1(x, W1, dis16)
  acc1 = _sc_scatter(y1, src2, dst2)
  y2 = _tc_mid(acc1, y1, dis16, b1.reshape(1, D), W2)
  acc2 = _sc_scatter(y2, src2, dst2)
  out = _tc_final(acc2, y2, dis16, b2.reshape(1, D),
                  batch.reshape(NB, 1, BN), batch.reshape(N, 1),
                  Wl1, bl1.reshape(1, 512), Wl2, bl2.reshape(1, D))
  return out


# speedup vs baseline: 1.0851x; 1.0851x over previous
import functools

import jax
import jax.numpy as jnp
from jax import lax
from jax.experimental import pallas as pl
from jax.experimental.pallas import tpu as pltpu
from jax.experimental.pallas import tpu_sc as plsc

N = 10000
E = 320000
D = 128
G = 64

NC = 2
NS = 16
NW = NC * NS

EC = 125
ER = E // EC
RPW = ER // NW
KC = 8
TO = RPW // KC
EPW = E // NW
RPS0 = 632
RPS1 = N - 15 * RPS0

def _mesh():
  return plsc.VectorSubcoreMesh(
      core_axis_name="c", subcore_axis_name="s", num_cores=NC, num_subcores=NS
  )


def _sc_degree(dst3):
  zeros = jnp.zeros((EPW,), jnp.float32)

  @functools.partial(
      pl.kernel,
      out_type=jax.ShapeDtypeStruct((NW, N), jnp.float32),
      mesh=_mesh(),
      compiler_params=pltpu.CompilerParams(use_tc_tiling_on_sc=False,
                                           needs_layout_passes=False),
      scratch_types=[
          pltpu.VMEM((EPW,), jnp.int32),
          pltpu.VMEM((N,), jnp.float32),
      ],
  )
  def k(dst_h, zeros_h, out_h, idx_v, acc_v):
    c = lax.axis_index("c")
    s = lax.axis_index("s")
    w = c * NS + s
    pltpu.sync_copy(zeros_h, acc_v)
    pltpu.sync_copy(dst_h.at[w], idx_v)
    ones16 = jnp.ones((16,), jnp.float32)

    @pl.loop(0, EPW // 16)
    def _r(r):
      iv = idx_v[pl.ds(r * 16, 16)]
      plsc.addupdate_scatter(acc_v, [iv], ones16)

    pltpu.sync_copy(acc_v, out_h.at[w])

  return k(dst3, zeros)


def _sc_scatter(y, src2, dst2):
  zeros = jnp.zeros((RPS0, D), jnp.float32)

  @functools.partial(
      pl.kernel,
      out_type=jax.ShapeDtypeStruct((NC, N, D), jnp.float32),
      mesh=_mesh(),
      scratch_types=[
          pltpu.VMEM((2, KC, EC), jnp.int32),
          pltpu.VMEM((2, KC, EC), jnp.int32),
          pltpu.VMEM((EC, D), jnp.float32),
          pltpu.VMEM((EC, D), jnp.float32),
          pltpu.VMEM_SHARED((N, D), jnp.float32),
          pltpu.SemaphoreType.DMA,
          pltpu.SemaphoreType.DMA,
          pltpu.SemaphoreType.DMA,
          pltpu.SemaphoreType.DMA,
          pltpu.SemaphoreType.DMA,
          pltpu.SemaphoreType.DMA,
      ],
  )
  def k(y_h, src_h, dst_h, zeros_h, out_h, sidx_v, didx_v,
        rows0_v, rows1_v, acc_sh,
        gsem0, gsem1, ssem0, ssem1,
        isem0, isem1):
    c = lax.axis_index("c")
    s = lax.axis_index("s")
    w = c * NS + s

    @pl.when(s < NS - 1)
    def _():
      pltpu.sync_copy(zeros_h.at[pl.ds(0, RPS0)], acc_sh.at[pl.ds(s * RPS0, RPS0)])
    @pl.when(s == NS - 1)
    def _():
      pltpu.sync_copy(zeros_h.at[pl.ds(0, RPS1)], acc_sh.at[pl.ds(15 * RPS0, RPS1)])
    plsc.subcore_barrier()

    base = w * RPW
    rows = (rows0_v, rows1_v)
    gsem = (gsem0, gsem1)
    ssem = (ssem0, ssem1)
    isem = (isem0, isem1)
    LA = 1

    def prefetch_idx(t, slot):
      pltpu.async_copy(src_h.at[pl.ds(base + t * KC, KC)],
                       sidx_v.at[slot], isem[slot])
      pltpu.async_copy(dst_h.at[pl.ds(base + t * KC, KC)],
                       didx_v.at[slot], isem[slot])

    def wait_idx(slot):
      pltpu.make_async_copy(src_h.at[pl.ds(base, KC)], sidx_v.at[slot],
                            isem[slot]).wait()
      pltpu.make_async_copy(dst_h.at[pl.ds(base, KC)], didx_v.at[slot],
                            isem[slot]).wait()

    def process(t, slot):
      @pl.when(t + 1 < TO)
      def _():
        prefetch_idx(t + 1, slot ^ 1)
      wait_idx(slot)
      si = sidx_v.at[slot]
      di = didx_v.at[slot]
      g = [None] * KC
      sd = [None] * KC
      for jj in range(LA):
        g[jj] = pltpu.async_copy(y_h.at[si.at[jj]], rows[jj & 1], gsem[jj & 1])
      for j in range(KC):
        if j + LA < KC:
          if j >= 1:
            sd[j - 1].wait()
          g[j + LA] = pltpu.async_copy(
              y_h.at[si.at[j + LA]], rows[(j + LA) & 1], gsem[(j + LA) & 1])
        g[j].wait()
        sd[j] = pltpu.async_copy(
            rows[j & 1], acc_sh.at[di.at[j]], ssem[j & 1], add=True)
      for j in range(KC - LA - 1, KC):
        sd[j].wait()

    prefetch_idx(0, 0)

    @pl.loop(0, TO // 2)
    def _outer(u):
      process(2 * u, 0)
      process(2 * u + 1, 1)

    if TO % 2 == 1:
      process(TO - 1, 0)

    plsc.subcore_barrier()

    @pl.when(s < NS - 1)
    def _():
      pltpu.sync_copy(acc_sh.at[pl.ds(s * RPS0, RPS0)],
                      out_h.at[c, pl.ds(s * RPS0, RPS0)])
    @pl.when(s == NS - 1)
    def _():
      pltpu.sync_copy(acc_sh.at[pl.ds(15 * RPS0, RPS1)],
                      out_h.at[c, pl.ds(15 * RPS0, RPS1)])

  return k(y, src2, dst2, zeros)


BN = 1000
NB = N // BN


def _tc_dis(degp):
  def body(d_ref, dis_ref):
    ones = jnp.ones((NW, 16), jnp.float32)
    deg = lax.dot_general(d_ref[...], ones, (((0,), (0,)), ((), ())),
                          preferred_element_type=jnp.float32) + 1.0
    dis_ref[...] = lax.rsqrt(deg)

  return pl.pallas_call(
      body,
      grid=(),
      in_specs=[pl.BlockSpec((NW, N), lambda: (0, 0))],
      out_specs=pl.BlockSpec((N, 16), lambda: (0, 0)),
      out_shape=jax.ShapeDtypeStruct((N, 16), jnp.float32),
  )(degp)


def _tc_y1(x, W1, dis16):
  def body(x_ref, w_ref, dis_ref, y_ref):
    xw = jnp.dot(x_ref[...], w_ref[...], preferred_element_type=jnp.float32)
    y_ref[...] = xw * dis_ref[:, 0:1]

  return pl.pallas_call(
      body,
      grid=(NB,),
      in_specs=[
          pl.BlockSpec((BN, D), lambda i: (i, 0)),
          pl.BlockSpec((D, D), lambda i: (0, 0)),
          pl.BlockSpec((BN, 16), lambda i: (i, 0)),
      ],
      out_specs=pl.BlockSpec((BN, D), lambda i: (i, 0)),
      out_shape=jax.ShapeDtypeStruct((N, D), jnp.float32),
  )(x, W1, dis16)


def _tc_mid(accp, y, dis16, b, W2):
  def body(a_ref, y_ref, dis_ref, b_ref, w_ref, o_ref):
    dis = dis_ref[:, 0:1]
    h = jnp.maximum(dis * (a_ref[0] + a_ref[1] + y_ref[...]) + b_ref[...], 0.0)
    hw = jnp.dot(h, w_ref[...], preferred_element_type=jnp.float32)
    o_ref[...] = hw * dis

  return pl.pallas_call(
      body,
      grid=(NB,),
      in_specs=[
          pl.BlockSpec((NC, BN, D), lambda i: (0, i, 0)),
          pl.BlockSpec((BN, D), lambda i: (i, 0)),
          pl.BlockSpec((BN, 16), lambda i: (i, 0)),
          pl.BlockSpec((1, D), lambda i: (0, 0)),
          pl.BlockSpec((D, D), lambda i: (0, 0)),
      ],
      out_specs=pl.BlockSpec((BN, D), lambda i: (i, 0)),
      out_shape=jax.ShapeDtypeStruct((N, D), jnp.float32),
  )(accp, y, dis16, b, W2)


def _tc_final(accp, y, dis16, b, brow, bcol, Wl1, bl1, Wl2, bl2):
  def body(a_ref, y_ref, dis_ref, b_ref, br_ref, bc_ref,
           wl1_ref, bl1_ref, wl2_ref, bl2_ref, o_ref,
           gsum, gmax, cnt):
    i = pl.program_id(0)

    @pl.when(i == 0)
    def _():
      gsum[...] = jnp.zeros_like(gsum)
      cnt[...] = jnp.zeros_like(cnt)
      gmax[...] = jnp.full_like(gmax, -jnp.inf)

    dis = dis_ref[:, 0:1]
    h = jnp.maximum(dis * (a_ref[0] + a_ref[1] + y_ref[...]) + b_ref[...], 0.0)

    brv = br_ref[0]
    oh = (lax.broadcasted_iota(jnp.int32, (G, BN), 0)
          == jnp.broadcast_to(brv, (G, BN))).astype(jnp.float32)
    gsum[...] += jnp.dot(oh, h, preferred_element_type=jnp.float32)
    cnt[...] += jnp.broadcast_to(jnp.sum(oh, axis=1, keepdims=True), (G, D))

    bcv = bc_ref[...]
    bmin = jnp.min(bcv)
    bmax = jnp.max(bcv)
    for g in range(G):
      @pl.when(jnp.logical_and(bmin <= g, g <= bmax))
      def _(g=g):
        m = jnp.max(jnp.where(bcv == g, h, -jnp.inf), axis=0, keepdims=True)
        gmax[g:g + 1, :] = jnp.maximum(gmax[g:g + 1, :], m)

    @pl.when(i == NB - 1)
    def _():
      mean = gsum[...] / jnp.maximum(cnt[...], 1.0)
      pooled = jnp.concatenate([gmax[...], mean], axis=1)
      h1 = jnp.maximum(
          jnp.dot(pooled, wl1_ref[...], preferred_element_type=jnp.float32)
          + bl1_ref[...], 0.0)
      o_ref[...] = (jnp.dot(h1, wl2_ref[...],
                            preferred_element_type=jnp.float32)
                    + bl2_ref[...])

  return pl.pallas_call(
      body,
      grid=(NB,),
      in_specs=[
          pl.BlockSpec((NC, BN, D), lambda i: (0, i, 0)),
          pl.BlockSpec((BN, D), lambda i: (i, 0)),
          pl.BlockSpec((BN, 16), lambda i: (i, 0)),
          pl.BlockSpec((1, D), lambda i: (0, 0)),
          pl.BlockSpec((1, 1, BN), lambda i: (i, 0, 0)),
          pl.BlockSpec((BN, 1), lambda i: (i, 0)),
          pl.BlockSpec((2 * D, 512), lambda i: (0, 0)),
          pl.BlockSpec((1, 512), lambda i: (0, 0)),
          pl.BlockSpec((512, D), lambda i: (0, 0)),
          pl.BlockSpec((1, D), lambda i: (0, 0)),
      ],
      out_specs=pl.BlockSpec((G, D), lambda i: (0, 0)),
      out_shape=jax.ShapeDtypeStruct((G, D), jnp.float32),
      scratch_shapes=[
          pltpu.VMEM((G, D), jnp.float32),
          pltpu.VMEM((G, D), jnp.float32),
          pltpu.VMEM((G, D), jnp.float32),
      ],
  )(accp, y, dis16, b, brow, bcol, Wl1, bl1, Wl2, bl2)


def kernel(x, edge_index, batch, W1, b1, W2, b2, Wl1, bl1, Wl2, bl2):
  src2 = edge_index[0].reshape(ER, EC)
  dst2 = edge_index[1].reshape(ER, EC)
  dst3 = edge_index[1].reshape(NW, EPW)

  degp = _sc_degree(dst3)
  dis16 = _tc_dis(degp)
  y1 = _tc_y1(x, W1, dis16)
  acc1 = _sc_scatter(y1, src2, dst2)
  y2 = _tc_mid(acc1, y1, dis16, b1.reshape(1, D), W2)
  acc2 = _sc_scatter(y2, src2, dst2)
  out = _tc_final(acc2, y2, dis16, b2.reshape(1, D),
                  batch.reshape(NB, 1, BN), batch.reshape(N, 1),
                  Wl1, bl1.reshape(1, 512), Wl2, bl2.reshape(1, D))
  return out


# speedup vs baseline: 1.1003x; 1.0140x over previous
import functools

import jax
import jax.numpy as jnp
from jax import lax
from jax.experimental import pallas as pl
from jax.experimental.pallas import tpu as pltpu
from jax.experimental.pallas import tpu_sc as plsc

N = 10000
E = 320000
D = 128
G = 64

NC = 2
NS = 16
NW = NC * NS

EC = 125
ER = E // EC
RPW = ER // NW
KC = 8
TO = RPW // KC
EPW = E // NW
RPS0 = 632
RPS1 = N - 15 * RPS0

def _mesh():
  return plsc.VectorSubcoreMesh(
      core_axis_name="c", subcore_axis_name="s", num_cores=NC, num_subcores=NS
  )


def _sc_degree(dst3):
  zeros = jnp.zeros((EPW,), jnp.float32)

  @functools.partial(
      pl.kernel,
      out_type=jax.ShapeDtypeStruct((NW, N), jnp.float32),
      mesh=_mesh(),
      compiler_params=pltpu.CompilerParams(use_tc_tiling_on_sc=False,
                                           needs_layout_passes=False),
      scratch_types=[
          pltpu.VMEM((EPW,), jnp.int32),
          pltpu.VMEM((N,), jnp.float32),
      ],
  )
  def k(dst_h, zeros_h, out_h, idx_v, acc_v):
    c = lax.axis_index("c")
    s = lax.axis_index("s")
    w = c * NS + s
    pltpu.sync_copy(zeros_h, acc_v)
    pltpu.sync_copy(dst_h.at[w], idx_v)
    ones16 = jnp.ones((16,), jnp.float32)

    @pl.loop(0, EPW // 16)
    def _r(r):
      iv = idx_v[pl.ds(r * 16, 16)]
      plsc.addupdate_scatter(acc_v, [iv], ones16)

    pltpu.sync_copy(acc_v, out_h.at[w])

  return k(dst3, zeros)


def _sc_scatter(y, src2, dst2):
  zeros = jnp.zeros((RPS0, D), jnp.float32)

  @functools.partial(
      pl.kernel,
      out_type=jax.ShapeDtypeStruct((NC, N, D), jnp.float32),
      mesh=_mesh(),
      scratch_types=[
          pltpu.VMEM((2, KC, EC), jnp.int32),
          pltpu.VMEM((2, KC, EC), jnp.int32),
          pltpu.VMEM((EC, D), jnp.float32),
          pltpu.VMEM((EC, D), jnp.float32),
          pltpu.VMEM_SHARED((N, D), jnp.float32),
          pltpu.SemaphoreType.DMA,
          pltpu.SemaphoreType.DMA,
          pltpu.SemaphoreType.DMA,
          pltpu.SemaphoreType.DMA,
          pltpu.SemaphoreType.DMA,
          pltpu.SemaphoreType.DMA,
      ],
  )
  def k(y_h, src_h, dst_h, zeros_h, out_h, sidx_v, didx_v,
        rows0_v, rows1_v, acc_sh,
        gsem0, gsem1, ssem0, ssem1,
        isem0, isem1):
    c = lax.axis_index("c")
    s = lax.axis_index("s")
    w = c * NS + s

    @pl.when(s < NS - 1)
    def _():
      pltpu.sync_copy(zeros_h.at[pl.ds(0, RPS0)], acc_sh.at[pl.ds(s * RPS0, RPS0)])
    @pl.when(s == NS - 1)
    def _():
      pltpu.sync_copy(zeros_h.at[pl.ds(0, RPS1)], acc_sh.at[pl.ds(15 * RPS0, RPS1)])
    plsc.subcore_barrier()

    base = w * RPW
    rows = (rows0_v, rows1_v)
    gsem = (gsem0, gsem1)
    ssem = (ssem0, ssem1)
    isem = (isem0, isem1)
    LA = 1

    def prefetch_idx(t, slot):
      pltpu.async_copy(src_h.at[pl.ds(base + t * KC, KC)],
                       sidx_v.at[slot], isem[slot])
      pltpu.async_copy(dst_h.at[pl.ds(base + t * KC, KC)],
                       didx_v.at[slot], isem[slot])

    def wait_idx(slot):
      pltpu.make_async_copy(src_h.at[pl.ds(base, KC)], sidx_v.at[slot],
                            isem[slot]).wait()
      pltpu.make_async_copy(dst_h.at[pl.ds(base, KC)], didx_v.at[slot],
                            isem[slot]).wait()

    def process(t, slot):
      @pl.when(t + 1 < TO)
      def _():
        prefetch_idx(t + 1, slot ^ 1)
      wait_idx(slot)
      si = sidx_v.at[slot]
      di = didx_v.at[slot]
      g = [None] * KC
      sd = [None] * KC
      for jj in range(LA):
        g[jj] = pltpu.async_copy(y_h.at[si.at[jj]], rows[jj & 1], gsem[jj & 1])
      for j in range(KC):
        if j + LA < KC:
          if j >= 1:
            sd[j - 1].wait()
          g[j + LA] = pltpu.async_copy(
              y_h.at[si.at[j + LA]], rows[(j + LA) & 1], gsem[(j + LA) & 1])
        g[j].wait()
        sd[j] = pltpu.async_copy(
            rows[j & 1], acc_sh.at[di.at[j]], ssem[j & 1], add=True)
      for j in range(KC - LA - 1, KC):
        sd[j].wait()

    prefetch_idx(0, 0)

    @pl.loop(0, TO // 2)
    def _outer(u):
      process(2 * u, 0)
      process(2 * u + 1, 1)

    if TO % 2 == 1:
      process(TO - 1, 0)

    plsc.subcore_barrier()

    @pl.when(s < NS - 1)
    def _():
      pltpu.sync_copy(acc_sh.at[pl.ds(s * RPS0, RPS0)],
                      out_h.at[c, pl.ds(s * RPS0, RPS0)])
    @pl.when(s == NS - 1)
    def _():
      pltpu.sync_copy(acc_sh.at[pl.ds(15 * RPS0, RPS1)],
                      out_h.at[c, pl.ds(15 * RPS0, RPS1)])

  return k(y, src2, dst2, zeros)


BN = 1000
NB = N // BN


def _tc_y1(x, W1, degp):
  def body(x_ref, w_ref, d_ref, y_ref, dis_ref, dis_sc):
    i = pl.program_id(0)

    @pl.when(i == 0)
    def _():
      ones = jnp.ones((NW, 16), jnp.float32)
      deg = lax.dot_general(d_ref[...], ones, (((0,), (0,)), ((), ())),
                            preferred_element_type=jnp.float32) + 1.0
      dis_sc[...] = lax.rsqrt(deg)

    dis_blk = dis_sc[pl.ds(i * BN, BN), :]
    dis_ref[...] = dis_blk
    xw = jnp.dot(x_ref[...], w_ref[...], preferred_element_type=jnp.float32)
    y_ref[...] = xw * dis_blk[:, 0:1]

  return pl.pallas_call(
      body,
      grid=(NB,),
      in_specs=[
          pl.BlockSpec((BN, D), lambda i: (i, 0)),
          pl.BlockSpec((D, D), lambda i: (0, 0)),
          pl.BlockSpec((NW, N), lambda i: (0, 0)),
      ],
      out_specs=[
          pl.BlockSpec((BN, D), lambda i: (i, 0)),
          pl.BlockSpec((BN, 16), lambda i: (i, 0)),
      ],
      out_shape=[
          jax.ShapeDtypeStruct((N, D), jnp.float32),
          jax.ShapeDtypeStruct((N, 16), jnp.float32),
      ],
      scratch_shapes=[pltpu.VMEM((N, 16), jnp.float32)],
  )(x, W1, degp)


def _tc_mid(accp, y, dis16, b, W2):
  def body(a_ref, y_ref, dis_ref, b_ref, w_ref, o_ref):
    dis = dis_ref[:, 0:1]
    h = jnp.maximum(dis * (a_ref[0] + a_ref[1] + y_ref[...]) + b_ref[...], 0.0)
    hw = jnp.dot(h, w_ref[...], preferred_element_type=jnp.float32)
    o_ref[...] = hw * dis

  return pl.pallas_call(
      body,
      grid=(NB,),
      in_specs=[
          pl.BlockSpec((NC, BN, D), lambda i: (0, i, 0)),
          pl.BlockSpec((BN, D), lambda i: (i, 0)),
          pl.BlockSpec((BN, 16), lambda i: (i, 0)),
          pl.BlockSpec((1, D), lambda i: (0, 0)),
          pl.BlockSpec((D, D), lambda i: (0, 0)),
      ],
      out_specs=pl.BlockSpec((BN, D), lambda i: (i, 0)),
      out_shape=jax.ShapeDtypeStruct((N, D), jnp.float32),
  )(accp, y, dis16, b, W2)


def _tc_final(accp, y, dis16, b, brow, bcol, Wl1, bl1, Wl2, bl2):
  def body(a_ref, y_ref, dis_ref, b_ref, br_ref, bc_ref,
           wl1_ref, bl1_ref, wl2_ref, bl2_ref, o_ref,
           gsum, gmax, cnt):
    i = pl.program_id(0)

    @pl.when(i == 0)
    def _():
      gsum[...] = jnp.zeros_like(gsum)
      cnt[...] = jnp.zeros_like(cnt)
      gmax[...] = jnp.full_like(gmax, -jnp.inf)

    dis = dis_ref[:, 0:1]
    h = jnp.maximum(dis * (a_ref[0] + a_ref[1] + y_ref[...]) + b_ref[...], 0.0)

    brv = br_ref[0]
    oh = (lax.broadcasted_iota(jnp.int32, (G, BN), 0)
          == jnp.broadcast_to(brv, (G, BN))).astype(jnp.float32)
    gsum[...] += jnp.dot(oh, h, preferred_element_type=jnp.float32)
    cnt[...] += jnp.broadcast_to(jnp.sum(oh, axis=1, keepdims=True), (G, D))

    bcv = bc_ref[...]
    bmin = jnp.min(bcv)
    bmax = jnp.max(bcv)
    for g in range(G):
      @pl.when(jnp.logical_and(bmin <= g, g <= bmax))
      def _(g=g):
        m = jnp.max(jnp.where(bcv == g, h, -jnp.inf), axis=0, keepdims=True)
        gmax[g:g + 1, :] = jnp.maximum(gmax[g:g + 1, :], m)

    @pl.when(i == NB - 1)
    def _():
      mean = gsum[...] / jnp.maximum(cnt[...], 1.0)
      pooled = jnp.concatenate([gmax[...], mean], axis=1)
      h1 = jnp.maximum(
          jnp.dot(pooled, wl1_ref[...], preferred_element_type=jnp.float32)
          + bl1_ref[...], 0.0)
      o_ref[...] = (jnp.dot(h1, wl2_ref[...],
                            preferred_element_type=jnp.float32)
                    + bl2_ref[...])

  return pl.pallas_call(
      body,
      grid=(NB,),
      in_specs=[
          pl.BlockSpec((NC, BN, D), lambda i: (0, i, 0)),
          pl.BlockSpec((BN, D), lambda i: (i, 0)),
          pl.BlockSpec((BN, 16), lambda i: (i, 0)),
          pl.BlockSpec((1, D), lambda i: (0, 0)),
          pl.BlockSpec((1, 1, BN), lambda i: (i, 0, 0)),
          pl.BlockSpec((BN, 1), lambda i: (i, 0)),
          pl.BlockSpec((2 * D, 512), lambda i: (0, 0)),
          pl.BlockSpec((1, 512), lambda i: (0, 0)),
          pl.BlockSpec((512, D), lambda i: (0, 0)),
          pl.BlockSpec((1, D), lambda i: (0, 0)),
      ],
      out_specs=pl.BlockSpec((G, D), lambda i: (0, 0)),
      out_shape=jax.ShapeDtypeStruct((G, D), jnp.float32),
      scratch_shapes=[
          pltpu.VMEM((G, D), jnp.float32),
          pltpu.VMEM((G, D), jnp.float32),
          pltpu.VMEM((G, D), jnp.float32),
      ],
  )(accp, y, dis16, b, brow, bcol, Wl1, bl1, Wl2, bl2)


def kernel(x, edge_index, batch, W1, b1, W2, b2, Wl1, bl1, Wl2, bl2):
  src2 = edge_index[0].reshape(ER, EC)
  dst2 = edge_index[1].reshape(ER, EC)
  dst3 = edge_index[1].reshape(NW, EPW)

  degp = _sc_degree(dst3)
  y1, dis16 = _tc_y1(x, W1, degp)
  acc1 = _sc_scatter(y1, src2, dst2)
  y2 = _tc_mid(acc1, y1, dis16, b1.reshape(1, D), W2)
  acc2 = _sc_scatter(y2, src2, dst2)
  out = _tc_final(acc2, y2, dis16, b2.reshape(1, D),
                  batch.reshape(NB, 1, BN), batch.reshape(N, 1),
                  Wl1, bl1.reshape(1, 512), Wl2, bl2.reshape(1, D))
  return out
